# baseline (speedup 1.0000x reference)
import functools

import jax
import jax.numpy as jnp
from jax.experimental import pallas as pl
from jax.experimental.pallas import tpu as pltpu

_CR = 8


def _stream_gather_kernel(labels_ref, drop_ref, table_ref, out_ref,
                          tbl3, out3, eff, bitmap, sem_in, sem_out,
                          *, batch, uncond):
    v, h = table_ref.shape
    nch = v // _CR
    tail = v - nch * _CR
    num_classes = v - 1

    for c in range(uncond):
        pltpu.make_async_copy(
            table_ref.at[pl.ds(c * _CR, _CR), :],
            tbl3.at[pl.ds(c * _CR, _CR), 0, :],
            sem_in,
        ).start()
    if tail:
        pltpu.make_async_copy(
            table_ref.at[pl.ds(nch * _CR, tail), :],
            tbl3.at[pl.ds(nch * _CR, tail), 0, :],
            sem_in,
        ).start()

    for c in range(nch):
        bitmap[c] = 0

    for i in range(batch):
        row = jnp.where(drop_ref[i] == 1, num_classes, labels_ref[i])
        row = jnp.clip(row, 0, num_classes)
        eff[i] = row
        bitmap[jnp.minimum(row // _CR, nch - 1)] = 1

    n_extra = jnp.int32(0)
    for c in range(uncond, nch):
        hit = bitmap[c]

        @pl.when(hit == 1)
        def _():
            pltpu.make_async_copy(
                table_ref.at[pl.ds(c * _CR, _CR), :],
                tbl3.at[pl.ds(c * _CR, _CR), 0, :],
                sem_in,
            ).start()

        n_extra = n_extra + hit

    n_rows = uncond * _CR + tail + n_extra * _CR
    pltpu.make_async_copy(
        tbl3.at[pl.ds(0, n_rows), 0, :],
        tbl3.at[pl.ds(0, n_rows), 0, :],
        sem_in,
    ).wait()

    q = batch // 8
    for s in range(8):
        lo = s * q
        hi = batch if s == 7 else (s + 1) * q
        for i in range(lo, hi):
            out3[i, 0] = tbl3[eff[i], 0]
        pltpu.make_async_copy(
            out3.at[pl.ds(lo, hi - lo), 0, :],
            out_ref.at[pl.ds(lo, hi - lo), :],
            sem_out,
        ).start()
    pltpu.make_async_copy(out3.at[:, 0, :], out_ref, sem_out).wait()


def kernel(labels, table, force_drop_ids):
    B = labels.shape[0]
    V, H = table.shape
    nch = V // _CR

    return pl.pallas_call(
        functools.partial(_stream_gather_kernel, batch=B,
                          uncond=max(1, nch // 4)),
        in_specs=[
            pl.BlockSpec(memory_space=pltpu.SMEM),
            pl.BlockSpec(memory_space=pltpu.SMEM),
            pl.BlockSpec(memory_space=pltpu.HBM),
        ],
        out_specs=pl.BlockSpec(memory_space=pltpu.HBM),
        out_shape=jax.ShapeDtypeStruct((B, H), table.dtype),
        scratch_shapes=[
            pltpu.VMEM((V, 1, H), table.dtype),
            pltpu.VMEM((B, 1, H), table.dtype),
            pltpu.SMEM((B,), jnp.int32),
            pltpu.SMEM((max(nch, 1),), jnp.int32),
            pltpu.SemaphoreType.DMA,
            pltpu.SemaphoreType.DMA,
        ],
        compiler_params=pltpu.CompilerParams(
            disable_bounds_checks=True,
        ),
    )(labels.astype(jnp.int32), force_drop_ids.astype(jnp.int32), table)


# speedup vs baseline: 1.3788x; 1.3788x over previous
import functools

import jax
import jax.numpy as jnp
from jax.experimental import pallas as pl
from jax.experimental.pallas import tpu as pltpu


def _stream_gather_kernel(labels_ref, drop_ref, table_ref, out_ref,
                          tbl3, out3, eff, sem_in, sem_out,
                          *, batch, n_chunks):
    v, h = table_ref.shape
    vc = v // n_chunks
    tail = v - n_chunks * vc
    for c in range(n_chunks):
        pltpu.make_async_copy(
            table_ref.at[pl.ds(c * vc, vc), :],
            tbl3.at[pl.ds(c * vc, vc), 0, :],
            sem_in,
        ).start()
    if tail:
        pltpu.make_async_copy(
            table_ref.at[pl.ds(n_chunks * vc, tail), :],
            tbl3.at[pl.ds(n_chunks * vc, tail), 0, :],
            sem_in,
        ).start()

    num_classes = v - 1
    for i in range(batch):
        row = jnp.where(drop_ref[i] == 1, num_classes, labels_ref[i])
        eff[i] = jnp.clip(row, 0, num_classes)

    pltpu.make_async_copy(table_ref, tbl3.at[:, 0, :], sem_in).wait()

    q = batch // 8
    for s in range(8):
        lo = s * q
        hi = batch if s == 7 else (s + 1) * q
        for i in range(lo, hi):
            out3[i, 0] = tbl3[eff[i], 0]
        pltpu.make_async_copy(
            out3.at[pl.ds(lo, hi - lo), 0, :],
            out_ref.at[pl.ds(lo, hi - lo), :],
            sem_out,
        ).start()
    pltpu.make_async_copy(out3.at[:, 0, :], out_ref, sem_out).wait()


def kernel(labels, table, force_drop_ids):
    B = labels.shape[0]
    V, H = table.shape

    return pl.pallas_call(
        functools.partial(_stream_gather_kernel, batch=B,
                          n_chunks=min(128, V)),
        in_specs=[
            pl.BlockSpec(memory_space=pltpu.SMEM),
            pl.BlockSpec(memory_space=pltpu.SMEM),
            pl.BlockSpec(memory_space=pltpu.HBM),
        ],
        out_specs=pl.BlockSpec(memory_space=pltpu.HBM),
        out_shape=jax.ShapeDtypeStruct((B, H), table.dtype),
        scratch_shapes=[
            pltpu.VMEM((V, 1, H), table.dtype),
            pltpu.VMEM((B, 1, H), table.dtype),
            pltpu.SMEM((B,), jnp.int32),
            pltpu.SemaphoreType.DMA,
            pltpu.SemaphoreType.DMA,
        ],
        compiler_params=pltpu.CompilerParams(
            disable_bounds_checks=True,
        ),
    )(labels.astype(jnp.int32), force_drop_ids.astype(jnp.int32), table)


# speedup vs baseline: 1.3823x; 1.0025x over previous
import functools

import jax
import jax.numpy as jnp
from jax.experimental import pallas as pl
from jax.experimental.pallas import tpu as pltpu


def _stream_gather_kernel(labels_ref, drop_ref, table_ref, out_ref,
                          tbl3, out3, eff, sem_in, sem_in2, sem_out,
                          *, batch, n_chunks):
    v, h = table_ref.shape
    vc = v // n_chunks
    tail = v - n_chunks * vc
    for c in range(n_chunks):
        pltpu.make_async_copy(
            table_ref.at[pl.ds(c * vc, vc), :],
            tbl3.at[pl.ds(c * vc, vc), 0, :],
            sem_in if c % 2 == 0 else sem_in2,
        ).start()
    if tail:
        pltpu.make_async_copy(
            table_ref.at[pl.ds(n_chunks * vc, tail), :],
            tbl3.at[pl.ds(n_chunks * vc, tail), 0, :],
            sem_in,
        ).start()

    num_classes = v - 1
    for i in range(batch):
        row = jnp.where(drop_ref[i] == 1, num_classes, labels_ref[i])
        eff[i] = jnp.clip(row, 0, num_classes)

    rows_even = ((n_chunks + 1) // 2) * vc + tail
    rows_odd = (n_chunks // 2) * vc
    pltpu.make_async_copy(
        tbl3.at[pl.ds(0, rows_even), 0, :],
        tbl3.at[pl.ds(0, rows_even), 0, :],
        sem_in,
    ).wait()
    if rows_odd:
        pltpu.make_async_copy(
            tbl3.at[pl.ds(0, rows_odd), 0, :],
            tbl3.at[pl.ds(0, rows_odd), 0, :],
            sem_in2,
        ).wait()

    q = batch // 8
    for s in range(8):
        lo = s * q
        hi = batch if s == 7 else (s + 1) * q
        for i in range(lo, hi):
            out3[i, 0] = tbl3[eff[i], 0]
        pltpu.make_async_copy(
            out3.at[pl.ds(lo, hi - lo), 0, :],
            out_ref.at[pl.ds(lo, hi - lo), :],
            sem_out,
        ).start()
    pltpu.make_async_copy(out3.at[:, 0, :], out_ref, sem_out).wait()


def kernel(labels, table, force_drop_ids):
    B = labels.shape[0]
    V, H = table.shape

    return pl.pallas_call(
        functools.partial(_stream_gather_kernel, batch=B,
                          n_chunks=min(128, V)),
        in_specs=[
            pl.BlockSpec(memory_space=pltpu.SMEM),
            pl.BlockSpec(memory_space=pltpu.SMEM),
            pl.BlockSpec(memory_space=pltpu.HBM),
        ],
        out_specs=pl.BlockSpec(memory_space=pltpu.HBM),
        out_shape=jax.ShapeDtypeStruct((B, H), table.dtype),
        scratch_shapes=[
            pltpu.VMEM((V, 1, H), table.dtype),
            pltpu.VMEM((B, 1, H), table.dtype),
            pltpu.SMEM((B,), jnp.int32),
            pltpu.SemaphoreType.DMA,
            pltpu.SemaphoreType.DMA,
            pltpu.SemaphoreType.DMA,
        ],
        compiler_params=pltpu.CompilerParams(
            disable_bounds_checks=True,
        ),
    )(labels.astype(jnp.int32), force_drop_ids.astype(jnp.int32), table)
